# baseline (speedup 1.0000x reference)
import jax
import jax.numpy as jnp
from jax import lax
from jax.experimental import pallas as pl
from jax.experimental.pallas import tpu as pltpu

N = 320000
D = 128
S = 4096
B = 12800
W = 256
NB = N // B
NWIN = S // W
T = NB + 2 * NWIN
DA = D + 128


def _segment_body(bid_ref, vid_ref, fst_ref, vld_ref,
                  rows_ref, ids_ref, o_ref):
    t = pl.program_id(0)
    v = vid_ref[t]
    fstv = fst_ref[t]
    vld = vld_ref[t]

    ids = ids_ref[0, 0, :]
    local = ids - v * W
    iota = lax.broadcasted_iota(jnp.int32, (W, B), 0)
    oh = jnp.logical_and(iota == local[None, :], vld > 0)
    ohb = oh.astype(jnp.bfloat16)

    rows = rows_ref[...]
    hi = rows.astype(jnp.bfloat16)
    lo = (rows - hi.astype(jnp.float32)).astype(jnp.bfloat16)
    onecol = jnp.concatenate(
        [jnp.ones((B, 1), jnp.bfloat16), jnp.zeros((B, 127), jnp.bfloat16)],
        axis=1)
    hi_aug = jnp.concatenate([hi, onecol], axis=1)

    dn = (((1,), (0,)), ((), ()))
    contrib = lax.dot_general(ohb, hi_aug, dn,
                              preferred_element_type=jnp.float32)
    lo_c = lax.dot_general(ohb, lo, dn, preferred_element_type=jnp.float32)

    @pl.when(fstv == 1)
    def _():
        o_ref[...] = contrib
        o_ref[:, :D] += lo_c

    @pl.when(fstv == 0)
    def _():
        o_ref[...] += contrib
        o_ref[:, :D] += lo_c


def _segment_sums(x, idx):
    edges = jnp.arange(NWIN + 1, dtype=jnp.int32) * W
    ws = jnp.searchsorted(idx, edges, side="left").astype(jnp.int32)
    nonempty = ws[1:] > ws[:-1]
    fb = ws[:-1] // B
    lb = jnp.where(nonempty, (ws[1:] - 1) // B, fb)
    nb = jnp.where(nonempty, lb - fb + 1, 1)
    starts = jnp.concatenate([jnp.zeros((1,), jnp.int32),
                              jnp.cumsum(nb).astype(jnp.int32)])
    treal = starts[NWIN]

    t = jnp.arange(T, dtype=jnp.int32)
    v_t = jnp.clip(jnp.searchsorted(starts, t, side="right").astype(jnp.int32)
                   - 1, 0, NWIN - 1)
    o_t = t - starts[v_t]
    block_id = jnp.clip(jnp.where(nonempty[v_t], fb[v_t] + o_t, 0), 0, NB - 1)
    valid = (t < treal).astype(jnp.int32)
    first = jnp.logical_and(o_t == 0, t < treal).astype(jnp.int32)

    ids3 = idx.reshape(NB, 1, B)

    grid_spec = pltpu.PrefetchScalarGridSpec(
        num_scalar_prefetch=4,
        grid=(T,),
        in_specs=[
            pl.BlockSpec((B, D), lambda t, bid, vid, fst, vld: (bid[t], 0)),
            pl.BlockSpec((1, 1, B),
                         lambda t, bid, vid, fst, vld: (bid[t], 0, 0)),
        ],
        out_specs=[
            pl.BlockSpec((W, DA), lambda t, bid, vid, fst, vld: (vid[t], 0)),
        ],
    )
    (acc,) = pl.pallas_call(
        _segment_body,
        grid_spec=grid_spec,
        out_shape=[jax.ShapeDtypeStruct((S, DA), jnp.float32)],
    )(block_id, v_t, first, valid, x, ids3)
    return acc


def _divide_body(a_ref, o_ref):
    o_ref[...] = a_ref[:, :D] / jnp.maximum(a_ref[:, D:D + 1], 1.0)


def _finalize(acc):
    return pl.pallas_call(
        _divide_body,
        out_shape=jax.ShapeDtypeStruct((S, D), jnp.float32),
    )(acc)


@jax.jit
def kernel(node_embeddings, batch_indices):
    idx = batch_indices.astype(jnp.int32)
    acc = _segment_sums(node_embeddings, idx)
    return _finalize(acc)


# speedup vs baseline: 1.1338x; 1.1338x over previous
import jax
import jax.numpy as jnp
from jax import lax
from jax.experimental import pallas as pl
from jax.experimental.pallas import tpu as pltpu

N = 320000
D = 128
S = 4096
B = 6400
W = 128
NB = N // B
NWIN = S // W
T = NB + 2 * NWIN
DA = D + 128


def _segment_body(bid_ref, vid_ref, fst_ref, vld_ref,
                  rows_ref, ids_ref, o_ref):
    t = pl.program_id(0)
    v = vid_ref[t]
    fstv = fst_ref[t]
    vld = vld_ref[t]

    ids = ids_ref[0, 0, :]
    local = ids - v * W
    iota = lax.broadcasted_iota(jnp.int32, (W, B), 0)
    oh = jnp.logical_and(iota == local[None, :], vld > 0)
    ohb = oh.astype(jnp.bfloat16)

    rows = rows_ref[...]
    hi = rows.astype(jnp.bfloat16)
    lo = (rows - hi.astype(jnp.float32)).astype(jnp.bfloat16)
    onecol = jnp.concatenate(
        [jnp.ones((B, 1), jnp.bfloat16), jnp.zeros((B, 127), jnp.bfloat16)],
        axis=1)
    hi_aug = jnp.concatenate([hi, onecol], axis=1)

    dn = (((1,), (0,)), ((), ()))
    contrib = lax.dot_general(ohb, hi_aug, dn,
                              preferred_element_type=jnp.float32)
    lo_c = lax.dot_general(ohb, lo, dn, preferred_element_type=jnp.float32)

    @pl.when(fstv == 1)
    def _():
        o_ref[...] = contrib
        o_ref[:, :D] += lo_c

    @pl.when(fstv == 0)
    def _():
        o_ref[...] += contrib
        o_ref[:, :D] += lo_c


def _segment_sums(x, idx):
    edges = jnp.arange(NWIN + 1, dtype=jnp.int32) * W
    ws = jnp.searchsorted(idx, edges, side="left").astype(jnp.int32)
    nonempty = ws[1:] > ws[:-1]
    fb = ws[:-1] // B
    lb = jnp.where(nonempty, (ws[1:] - 1) // B, fb)
    nb = jnp.where(nonempty, lb - fb + 1, 1)
    starts = jnp.concatenate([jnp.zeros((1,), jnp.int32),
                              jnp.cumsum(nb).astype(jnp.int32)])
    treal = starts[NWIN]

    t = jnp.arange(T, dtype=jnp.int32)
    v_t = jnp.clip(jnp.searchsorted(starts, t, side="right").astype(jnp.int32)
                   - 1, 0, NWIN - 1)
    o_t = t - starts[v_t]
    block_id = jnp.clip(jnp.where(nonempty[v_t], fb[v_t] + o_t, 0), 0, NB - 1)
    valid = (t < treal).astype(jnp.int32)
    first = jnp.logical_and(o_t == 0, t < treal).astype(jnp.int32)

    ids3 = idx.reshape(NB, 1, B)

    grid_spec = pltpu.PrefetchScalarGridSpec(
        num_scalar_prefetch=4,
        grid=(T,),
        in_specs=[
            pl.BlockSpec((B, D), lambda t, bid, vid, fst, vld: (bid[t], 0)),
            pl.BlockSpec((1, 1, B),
                         lambda t, bid, vid, fst, vld: (bid[t], 0, 0)),
        ],
        out_specs=[
            pl.BlockSpec((W, DA), lambda t, bid, vid, fst, vld: (vid[t], 0)),
        ],
    )
    (acc,) = pl.pallas_call(
        _segment_body,
        grid_spec=grid_spec,
        out_shape=[jax.ShapeDtypeStruct((S, DA), jnp.float32)],
    )(block_id, v_t, first, valid, x, ids3)
    return acc


def _divide_body(a_ref, o_ref):
    o_ref[...] = a_ref[:, :D] / jnp.maximum(a_ref[:, D:D + 1], 1.0)


def _finalize(acc):
    return pl.pallas_call(
        _divide_body,
        out_shape=jax.ShapeDtypeStruct((S, D), jnp.float32),
    )(acc)


@jax.jit
def kernel(node_embeddings, batch_indices):
    idx = batch_indices.astype(jnp.int32)
    acc = _segment_sums(node_embeddings, idx)
    return _finalize(acc)


# speedup vs baseline: 1.1788x; 1.0396x over previous
import jax
import jax.numpy as jnp
from jax import lax
from jax.experimental import pallas as pl
from jax.experimental.pallas import tpu as pltpu

N = 320000
D = 128
S = 4096
B = 6400
W = 128
NB = N // B
NWIN = S // W
T = NB + 2 * NWIN
DA = D + 128


def _segment_body(bid_ref, vid_ref, fst_ref, vld_ref,
                  rows_ref, ids_ref, o_ref):
    t = pl.program_id(0)
    v = vid_ref[t]
    fstv = fst_ref[t]
    vld = vld_ref[t]

    ids = ids_ref[0, 0, :]
    local = ids - v * W
    iota = lax.broadcasted_iota(jnp.int32, (W, B), 0)
    oh = jnp.logical_and(iota == local[None, :], vld > 0)
    ohb = oh.astype(jnp.bfloat16)

    rows = rows_ref[...]
    hi = rows.astype(jnp.bfloat16)
    onecol = jnp.concatenate(
        [jnp.ones((B, 1), jnp.bfloat16), jnp.zeros((B, 127), jnp.bfloat16)],
        axis=1)
    hi_aug = jnp.concatenate([hi, onecol], axis=1)

    dn = (((1,), (0,)), ((), ()))
    contrib = lax.dot_general(ohb, hi_aug, dn,
                              preferred_element_type=jnp.float32)

    @pl.when(fstv == 1)
    def _():
        o_ref[...] = contrib

    @pl.when(fstv == 0)
    def _():
        o_ref[...] += contrib


def _segment_sums(x, idx):
    edges = jnp.arange(NWIN + 1, dtype=jnp.int32) * W
    ws = jnp.searchsorted(idx, edges, side="left").astype(jnp.int32)
    nonempty = ws[1:] > ws[:-1]
    fb = ws[:-1] // B
    lb = jnp.where(nonempty, (ws[1:] - 1) // B, fb)
    nb = jnp.where(nonempty, lb - fb + 1, 1)
    starts = jnp.concatenate([jnp.zeros((1,), jnp.int32),
                              jnp.cumsum(nb).astype(jnp.int32)])
    treal = starts[NWIN]

    t = jnp.arange(T, dtype=jnp.int32)
    v_t = jnp.clip(jnp.searchsorted(starts, t, side="right").astype(jnp.int32)
                   - 1, 0, NWIN - 1)
    o_t = t - starts[v_t]
    block_id = jnp.clip(jnp.where(nonempty[v_t], fb[v_t] + o_t, 0), 0, NB - 1)
    valid = (t < treal).astype(jnp.int32)
    first = jnp.logical_and(o_t == 0, t < treal).astype(jnp.int32)

    ids3 = idx.reshape(NB, 1, B)

    grid_spec = pltpu.PrefetchScalarGridSpec(
        num_scalar_prefetch=4,
        grid=(T,),
        in_specs=[
            pl.BlockSpec((B, D), lambda t, bid, vid, fst, vld: (bid[t], 0)),
            pl.BlockSpec((1, 1, B),
                         lambda t, bid, vid, fst, vld: (bid[t], 0, 0)),
        ],
        out_specs=[
            pl.BlockSpec((W, DA), lambda t, bid, vid, fst, vld: (vid[t], 0)),
        ],
    )
    (acc,) = pl.pallas_call(
        _segment_body,
        grid_spec=grid_spec,
        out_shape=[jax.ShapeDtypeStruct((S, DA), jnp.float32)],
    )(block_id, v_t, first, valid, x, ids3)
    return acc


def _divide_body(a_ref, o_ref):
    o_ref[...] = a_ref[:, :D] / jnp.maximum(a_ref[:, D:D + 1], 1.0)


def _finalize(acc):
    return pl.pallas_call(
        _divide_body,
        out_shape=jax.ShapeDtypeStruct((S, D), jnp.float32),
    )(acc)


@jax.jit
def kernel(node_embeddings, batch_indices):
    idx = batch_indices.astype(jnp.int32)
    acc = _segment_sums(node_embeddings, idx)
    return _finalize(acc)
